# Initial kernel scaffold; baseline (speedup 1.0000x reference)
import jax
import jax.numpy as jnp
from jax.experimental import pallas as pl


def kernel(x, edge_index, edge_attr, batch, We, be, W1, b1, W2, b2, gamma, beta):
    raise NotImplementedError("write your pallas kernel here")


# speedup vs baseline: 2.0561x; 2.0561x over previous
import functools

import jax
import jax.numpy as jnp
from jax import lax
from jax.experimental import pallas as pl
from jax.experimental.pallas import tpu as pltpu
from jax.experimental.pallas import tpu_sc as plsc

N_NODES = 10000
N_EDGES = 320000
EMB = 128
D_EDGE = 16
N_GRAPHS = 32

NC = 2
NS = 16
NW = NC * NS
CHUNK = 128

GS_CHUNKS = 78

ROWS_PER_TILE = 624
ROWS_LAST = N_NODES - 15 * ROWS_PER_TILE
ZROWS = 104

_mesh = plsc.VectorSubcoreMesh(core_axis_name="c", subcore_axis_name="s")


def _zero_acc_slice(zbuf, acc, s):
    base_r = s * ROWS_PER_TILE
    for r in range(ROWS_PER_TILE // ZROWS):
        pltpu.sync_copy(zbuf, acc.at[pl.ds(base_r + r * ZROWS, ZROWS)])

    @pl.when(s == NS - 1)
    def _():
        pltpu.sync_copy(zbuf.at[pl.ds(0, ROWS_LAST - ROWS_PER_TILE)],
                        acc.at[pl.ds(base_r + ROWS_PER_TILE, ROWS_LAST - ROWS_PER_TILE)])


def _write_acc_slice(acc, out_hbm, s):
    base_r = s * ROWS_PER_TILE
    pltpu.sync_copy(acc.at[pl.ds(base_r, ROWS_PER_TILE)],
                    out_hbm.at[pl.ds(base_r, ROWS_PER_TILE)])

    @pl.when(s == NS - 1)
    def _():
        pltpu.sync_copy(acc.at[pl.ds(base_r + ROWS_PER_TILE, ROWS_LAST - ROWS_PER_TILE)],
                        out_hbm.at[pl.ds(base_r + ROWS_PER_TILE, ROWS_LAST - ROWS_PER_TILE)])


@functools.partial(
    pl.kernel,
    mesh=_mesh,
    out_type=jax.ShapeDtypeStruct((NC, N_NODES, EMB), jnp.float32),
    scratch_types=[
        pltpu.VMEM((CHUNK,), jnp.int32),
        pltpu.VMEM((CHUNK,), jnp.int32),
        pltpu.VMEM((CHUNK, EMB), jnp.float32),
        pltpu.VMEM((CHUNK, EMB), jnp.float32),
        pltpu.VMEM((ZROWS, EMB), jnp.float32),
        pltpu.VMEM_SHARED((N_NODES, EMB), jnp.float32),
        pltpu.SemaphoreType.DMA,
    ],
)
def _adj_gs(h_hbm, e_hbm, src_hbm, dst_hbm, z_hbm, out_hbm,
            src_v, dst_v, rows_v, erows_v, zbuf, acc, sem):
    c = lax.axis_index("c")
    s = lax.axis_index("s")
    wid = s * NC + c

    pltpu.sync_copy(z_hbm, zbuf)
    _zero_acc_slice(zbuf, acc, s)
    plsc.subcore_barrier()

    nchunks = GS_CHUNKS + jnp.where(wid < 4, 1, 0)
    ebase = wid * (GS_CHUNKS * CHUNK) + jnp.minimum(wid, 4) * CHUNK

    def body(i, carry):
        off = pl.multiple_of(ebase + i * CHUNK, 8)
        pltpu.sync_copy(src_hbm.at[pl.ds(off, CHUNK)], src_v)
        pltpu.sync_copy(dst_hbm.at[pl.ds(off, CHUNK)], dst_v)
        pltpu.sync_copy(e_hbm.at[pl.ds(off, CHUNK)], erows_v)
        pltpu.async_copy(h_hbm.at[src_v], rows_v, sem).wait()
        pltpu.sync_copy(rows_v, acc.at[dst_v], add=True)
        pltpu.sync_copy(erows_v, acc.at[dst_v], add=True)
        return carry

    lax.fori_loop(0, nchunks, body, 0)
    plsc.subcore_barrier()
    _write_acc_slice(acc, out_hbm.at[c], s)


_EB = 2000


def _edge_e_body(ea_ref, we_ref, be_ref, e_ref):
    e_ref[...] = jnp.dot(ea_ref[...], we_ref[...],
                         preferred_element_type=jnp.float32) + be_ref[...]


_edge_e = pl.pallas_call(
    _edge_e_body,
    grid=(N_EDGES // _EB,),
    in_specs=[pl.BlockSpec((_EB, D_EDGE), lambda i: (i, 0)),
              pl.BlockSpec((D_EDGE, EMB), lambda i: (0, 0)),
              pl.BlockSpec((1, EMB), lambda i: (0, 0))],
    out_specs=pl.BlockSpec((_EB, EMB), lambda i: (i, 0)),
    out_shape=jax.ShapeDtypeStruct((N_EDGES, EMB), jnp.float32),
)

_HI = lax.Precision.HIGHEST


def _layer_body(h_ref, p_ref, w1_ref, b1_ref, w2_ref, b2_ref, g_ref, bt_ref,
                out_ref, *, last_relu):
    agg = p_ref[...] + h_ref[...]
    u = jnp.dot(agg, w1_ref[...],
                preferred_element_type=jnp.float32) + b1_ref[...]
    u = jnp.maximum(u, 0.0)
    v = jnp.dot(u, w2_ref[...],
                preferred_element_type=jnp.float32) + b2_ref[...]
    mu = jnp.mean(v, axis=0, keepdims=True)
    var = jnp.mean((v - mu) ** 2, axis=0, keepdims=True)
    v = (v - mu) / jnp.sqrt(var + 1e-5) * g_ref[...] + bt_ref[...]
    if last_relu:
        v = jnp.maximum(v, 0.0)
    out_ref[...] = v


def _pool_layer_body(h_ref, p_ref, w1_ref, b1_ref, w2_ref, b2_ref, g_ref,
                     bt_ref, batch_ref, out_ref):
    agg = p_ref[...] + h_ref[...]
    u = jnp.dot(agg, w1_ref[...],
                preferred_element_type=jnp.float32) + b1_ref[...]
    u = jnp.maximum(u, 0.0)
    v = jnp.dot(u, w2_ref[...],
                preferred_element_type=jnp.float32) + b2_ref[...]
    mu = jnp.mean(v, axis=0, keepdims=True)
    var = jnp.mean((v - mu) ** 2, axis=0, keepdims=True)
    v = (v - mu) / jnp.sqrt(var + 1e-5) * g_ref[...] + bt_ref[...]
    gid = lax.broadcasted_iota(jnp.int32, (1, N_GRAPHS), 1)
    m = (batch_ref[...] == gid).astype(jnp.float32)
    dn = (((0,), (0,)), ((), ()))
    seg = lax.dot_general(m, v, dn, preferred_element_type=jnp.float32,
                          precision=_HI)
    cnt = lax.dot_general(m, jnp.ones_like(v[:, 0:1]), dn,
                          preferred_element_type=jnp.float32,
                          precision=_HI)
    out_ref[...] = seg / jnp.maximum(cnt, 1.0)


_mid_layer = pl.pallas_call(
    functools.partial(_layer_body, last_relu=True),
    out_shape=jax.ShapeDtypeStruct((N_NODES, EMB), jnp.float32),
)

_pool_layer = pl.pallas_call(
    _pool_layer_body,
    out_shape=jax.ShapeDtypeStruct((N_GRAPHS, EMB), jnp.float32),
)


def kernel(x, edge_index, edge_attr, batch, We, be, W1, b1, W2, b2, gamma, beta):
    src = edge_index[0]
    dst = edge_index[1]
    zeros_nd = jnp.zeros((ZROWS, EMB), jnp.float32)
    batch2d = batch.reshape(N_NODES, 1)

    outs = []
    for l in range(2):
        h = x
        for k in range(3):
            e = _edge_e(edge_attr, We[l, k], be[l, k].reshape(1, EMB))
            pp = _adj_gs(h, e, src, dst, zeros_nd)
            p = pp[0] + pp[1]
            args = (h, p,
                    W1[l, k], b1[l, k].reshape(1, EMB),
                    W2[l, k], b2[l, k].reshape(1, EMB),
                    gamma[l, k].reshape(1, EMB), beta[l, k].reshape(1, EMB))
            if k < 2:
                h = _mid_layer(*args)
            else:
                outs.append(_pool_layer(*args, batch2d))
    return tuple(outs)
